# initial kernel scaffold (unmeasured)
import jax
import jax.numpy as jnp
from jax import lax
from jax.experimental import pallas as pl
from jax.experimental.pallas import tpu as pltpu

N_DEV = 4
E_LOC = 4
E_TOT = 16
T = 512
D = 256
H = 512


def kernel(x, router_W, route_idx, expert_W):
    def body(x_ref, rw_ref, idx_ref, ew_ref, out_ref,
             comm_ref, send_sems, recv_sems):
        my = lax.axis_index("i")
        left = lax.rem(my + (N_DEV - 1), N_DEV)
        right = lax.rem(my + 1, N_DEV)

        comm_ref[0, :, :, :] = ew_ref[:, :, :].astype(jnp.bfloat16)

        barrier_sem = pltpu.get_barrier_semaphore()
        for nbr in (left, right):
            pl.semaphore_signal(
                barrier_sem, inc=1,
                device_id=(nbr,), device_id_type=pl.DeviceIdType.MESH,
            )
        pl.semaphore_wait(barrier_sem, 2)

        scores = jnp.dot(x_ref[:, :], rw_ref[:, :],
                         preferred_element_type=jnp.float32)
        e0 = idx_ref[:, 0:1]
        e1 = idx_ref[:, 1:2]
        iota = lax.broadcasted_iota(jnp.int32, (T, E_TOT), 1)
        oh0 = iota == e0
        oh1 = iota == e1
        s0 = jnp.sum(jnp.where(oh0, scores, 0.0), axis=1, keepdims=True)
        s1 = jnp.sum(jnp.where(oh1, scores, 0.0), axis=1, keepdims=True)
        m = jnp.maximum(s0, s1)
        g0 = jnp.exp(s0 - m)
        g1 = jnp.exp(s1 - m)
        denom = g0 + g1
        w = (oh0 * (g0 / denom) + oh1 * (g1 / denom)).astype(jnp.float32)

        x_b = x_ref[:, :].astype(jnp.bfloat16)

        def contribution(slot, acc):
            origin = lax.rem(my + (N_DEV - slot), N_DEV)
            wblk = lax.dynamic_slice(w, (0, origin * E_LOC), (T, E_LOC))
            for k in range(E_LOC):
                h_k = jnp.dot(x_b, comm_ref[slot, k, :, :],
                              preferred_element_type=jnp.float32)
                acc = acc + wblk[:, k:k + 1] * h_k
            return acc

        acc = jnp.zeros((T, H), jnp.float32)
        rdmas = []
        for hop in range(N_DEV - 1):
            rdma = pltpu.make_async_remote_copy(
                src_ref=comm_ref.at[hop],
                dst_ref=comm_ref.at[hop + 1],
                send_sem=send_sems.at[hop],
                recv_sem=recv_sems.at[hop],
                device_id=(right,),
                device_id_type=pl.DeviceIdType.MESH,
            )
            rdma.start()
            rdmas.append(rdma)
            acc = contribution(hop, acc)
            rdmas[hop].wait_recv()
        acc = contribution(N_DEV - 1, acc)
        for rdma in rdmas:
            rdma.wait_send()
        out_ref[:, :] = acc

    return pl.pallas_call(
        body,
        out_shape=jax.ShapeDtypeStruct((T, H), jnp.float32),
        in_specs=[
            pl.BlockSpec(memory_space=pltpu.VMEM),
            pl.BlockSpec(memory_space=pltpu.VMEM),
            pl.BlockSpec(memory_space=pltpu.VMEM),
            pl.BlockSpec(memory_space=pltpu.VMEM),
        ],
        out_specs=pl.BlockSpec(memory_space=pltpu.VMEM),
        scratch_shapes=[
            pltpu.VMEM((N_DEV, E_LOC, D, H), jnp.bfloat16),
            pltpu.SemaphoreType.DMA((N_DEV - 1,)),
            pltpu.SemaphoreType.DMA((N_DEV - 1,)),
        ],
        compiler_params=pltpu.CompilerParams(collective_id=0),
    )(x, router_W, route_idx, expert_W)


# baseline (device time: 48400 ns/iter reference)
import jax
import jax.numpy as jnp
from jax import lax
from jax.experimental import pallas as pl
from jax.experimental.pallas import tpu as pltpu

N_DEV = 4
E_LOC = 4
E_TOT = 16
T = 512
D = 256
H = 512


def kernel(x, router_W, route_idx, expert_W):
    def body(x_ref, rw_ref, idx_ref, ew_ref, out_ref,
             comm_ref, send_sems, recv_sems):
        my = lax.axis_index("i")
        left = lax.rem(my + (N_DEV - 1), N_DEV)
        right = lax.rem(my + 1, N_DEV)

        comm_ref[0, :, :, :] = ew_ref[:, :, :].astype(jnp.bfloat16)

        barrier_sem = pltpu.get_barrier_semaphore()
        for nbr in (left, right):
            pl.semaphore_signal(
                barrier_sem, inc=1,
                device_id=(nbr,), device_id_type=pl.DeviceIdType.MESH,
            )
        pl.semaphore_wait(barrier_sem, 2)

        scores = jnp.dot(x_ref[:, :], rw_ref[:, :],
                         preferred_element_type=jnp.float32)
        e0 = idx_ref[:, 0:1]
        e1 = idx_ref[:, 1:2]
        iota = lax.broadcasted_iota(jnp.int32, (T, E_TOT), 1)
        oh0 = iota == e0
        oh1 = iota == e1
        s0 = jnp.sum(jnp.where(oh0, scores, 0.0), axis=1, keepdims=True)
        s1 = jnp.sum(jnp.where(oh1, scores, 0.0), axis=1, keepdims=True)
        m = jnp.maximum(s0, s1)
        g0 = jnp.exp(s0 - m)
        g1 = jnp.exp(s1 - m)
        denom = g0 + g1
        w = (oh0 * (g0 / denom) + oh1 * (g1 / denom)).astype(jnp.float32)

        x_b = x_ref[:, :].astype(jnp.bfloat16)

        def contribution(slot, acc):
            origin = lax.rem(my + (N_DEV - slot), N_DEV)
            for k in range(E_LOC):
                w_k = jnp.sum(
                    jnp.where(iota == origin * E_LOC + k, w, 0.0),
                    axis=1, keepdims=True)
                h_k = jnp.dot(x_b, comm_ref[slot, k, :, :],
                              preferred_element_type=jnp.float32)
                acc = acc + w_k * h_k
            return acc

        acc = jnp.zeros((T, H), jnp.float32)
        rdmas = []
        for hop in range(N_DEV - 1):
            rdma = pltpu.make_async_remote_copy(
                src_ref=comm_ref.at[hop],
                dst_ref=comm_ref.at[hop + 1],
                send_sem=send_sems.at[hop],
                recv_sem=recv_sems.at[hop],
                device_id=(right,),
                device_id_type=pl.DeviceIdType.MESH,
            )
            rdma.start()
            rdmas.append(rdma)
            acc = contribution(hop, acc)
            rdmas[hop].wait_recv()
        acc = contribution(N_DEV - 1, acc)
        for rdma in rdmas:
            rdma.wait_send()
        out_ref[:, :] = acc

    return pl.pallas_call(
        body,
        out_shape=jax.ShapeDtypeStruct((T, H), jnp.float32),
        in_specs=[
            pl.BlockSpec(memory_space=pltpu.VMEM),
            pl.BlockSpec(memory_space=pltpu.VMEM),
            pl.BlockSpec(memory_space=pltpu.VMEM),
            pl.BlockSpec(memory_space=pltpu.VMEM),
        ],
        out_specs=pl.BlockSpec(memory_space=pltpu.VMEM),
        scratch_shapes=[
            pltpu.VMEM((N_DEV, E_LOC, D, H), jnp.bfloat16),
            pltpu.SemaphoreType.DMA((N_DEV - 1,)),
            pltpu.SemaphoreType.DMA((N_DEV - 1,)),
        ],
        compiler_params=pltpu.CompilerParams(collective_id=0),
    )(x, router_W, route_idx, expert_W)


# device time: 31554 ns/iter; 1.5339x vs baseline; 1.5339x over previous
import jax
import jax.numpy as jnp
from jax import lax
from jax.experimental import pallas as pl
from jax.experimental.pallas import tpu as pltpu

N_DEV = 4
E_LOC = 4
E_HALF = 2
E_TOT = 16
T = 512
D = 256
H = 512


def kernel(x, router_W, route_idx, expert_W):
    def body(x_ref, rw_ref, idx_ref, ew_ref, out_ref,
             cw_ref, ccw_ref, cw_send, cw_recv, ccw_send, ccw_recv):
        my = lax.axis_index("i")
        left = lax.rem(my + (N_DEV - 1), N_DEV)
        right = lax.rem(my + 1, N_DEV)

        cw_ref[0, :, :, :] = ew_ref[0:E_HALF, :, :].astype(jnp.bfloat16)
        ccw_ref[0, :, :, :] = ew_ref[E_HALF:E_LOC, :, :].astype(jnp.bfloat16)

        barrier_sem = pltpu.get_barrier_semaphore()
        for nbr in (left, right):
            pl.semaphore_signal(
                barrier_sem, inc=1,
                device_id=(nbr,), device_id_type=pl.DeviceIdType.MESH,
            )
        pl.semaphore_wait(barrier_sem, 2)

        scores = jnp.dot(x_ref[:, :], rw_ref[:, :],
                         preferred_element_type=jnp.float32)
        e0 = idx_ref[:, 0:1]
        e1 = idx_ref[:, 1:2]
        iota = lax.broadcasted_iota(jnp.int32, (T, E_TOT), 1)
        oh0 = iota == e0
        oh1 = iota == e1
        s0 = jnp.sum(jnp.where(oh0, scores, 0.0), axis=1, keepdims=True)
        s1 = jnp.sum(jnp.where(oh1, scores, 0.0), axis=1, keepdims=True)
        m = jnp.maximum(s0, s1)
        g0 = jnp.exp(s0 - m)
        g1 = jnp.exp(s1 - m)
        denom = g0 + g1
        w = (oh0 * (g0 / denom) + oh1 * (g1 / denom)).astype(jnp.float32)

        x_b = x_ref[:, :].astype(jnp.bfloat16)

        def contribution(ref, slot, origin, k_base, acc):
            for k in range(E_HALF):
                w_k = jnp.sum(
                    jnp.where(iota == origin * E_LOC + k_base + k, w, 0.0),
                    axis=1, keepdims=True)
                h_k = jnp.dot(x_b, ref[slot, k, :, :],
                              preferred_element_type=jnp.float32)
                acc = acc + w_k * h_k
            return acc

        def slot_contribution(slot, acc):
            o_cw = lax.rem(my + (N_DEV - slot), N_DEV)
            o_ccw = lax.rem(my + slot, N_DEV)
            acc = contribution(cw_ref, slot, o_cw, 0, acc)
            acc = contribution(ccw_ref, slot, o_ccw, E_HALF, acc)
            return acc

        acc = jnp.zeros((T, H), jnp.float32)
        rdmas = []
        for hop in range(N_DEV - 1):
            r_cw = pltpu.make_async_remote_copy(
                src_ref=cw_ref.at[hop], dst_ref=cw_ref.at[hop + 1],
                send_sem=cw_send.at[hop], recv_sem=cw_recv.at[hop],
                device_id=(right,), device_id_type=pl.DeviceIdType.MESH,
            )
            r_ccw = pltpu.make_async_remote_copy(
                src_ref=ccw_ref.at[hop], dst_ref=ccw_ref.at[hop + 1],
                send_sem=ccw_send.at[hop], recv_sem=ccw_recv.at[hop],
                device_id=(left,), device_id_type=pl.DeviceIdType.MESH,
            )
            r_cw.start()
            r_ccw.start()
            rdmas += [r_cw, r_ccw]
            acc = slot_contribution(hop, acc)
            r_cw.wait_recv()
            r_ccw.wait_recv()
        acc = slot_contribution(N_DEV - 1, acc)
        for rdma in rdmas:
            rdma.wait_send()
        out_ref[:, :] = acc

    return pl.pallas_call(
        body,
        out_shape=jax.ShapeDtypeStruct((T, H), jnp.float32),
        in_specs=[
            pl.BlockSpec(memory_space=pltpu.VMEM),
            pl.BlockSpec(memory_space=pltpu.VMEM),
            pl.BlockSpec(memory_space=pltpu.VMEM),
            pl.BlockSpec(memory_space=pltpu.VMEM),
        ],
        out_specs=pl.BlockSpec(memory_space=pltpu.VMEM),
        scratch_shapes=[
            pltpu.VMEM((N_DEV, E_HALF, D, H), jnp.bfloat16),
            pltpu.VMEM((N_DEV, E_HALF, D, H), jnp.bfloat16),
            pltpu.SemaphoreType.DMA((N_DEV - 1,)),
            pltpu.SemaphoreType.DMA((N_DEV - 1,)),
            pltpu.SemaphoreType.DMA((N_DEV - 1,)),
            pltpu.SemaphoreType.DMA((N_DEV - 1,)),
        ],
        compiler_params=pltpu.CompilerParams(collective_id=0),
    )(x, router_W, route_idx, expert_W)
